# baseline (speedup 1.0000x reference)
import functools
import math

import jax
import jax.numpy as jnp
from jax import lax
from jax.experimental import pallas as pl
from jax.experimental.pallas import tpu as pltpu
from jax.experimental.pallas import tpu_sc as plsc

N = 10000
E = 160000
HEADS = 4
F_IN = 92
F_EDGE = 41
NH = 64
NL = 3
D_CAT = NH + NH
D_OUT = HEADS * NH

_GB = 128
E_PAD = 1280 * _GB
N_TAB = N + 16
PAD_ROW = N
EB = 2048
DENW = 16

_NC = 2
_NS = 16
_NW = _NC * _NS
_ROWS = E_PAD // _GB
_ROWS_W = _ROWS // _NW
_K = 4

_MESH = plsc.VectorSubcoreMesh(core_axis_name="c", subcore_axis_name="s")
_SC_PARAMS = pltpu.CompilerParams(use_tc_tiling_on_sc=False)


EB2 = 1024

import numpy as _np

_S_HALF = _np.zeros((D_OUT, DENW), _np.float32)
for _h in range(HEADS):
    _S_HALF[NH * _h:NH * (_h + 1), _h] = 1.0 / math.sqrt(NH)

_EXPAND = _np.zeros((2 * DENW, 2 * D_OUT), _np.float32)
_FOLD = _np.zeros((2 * D_OUT, D_CAT), _np.float32)
_AVG = _np.zeros((D_CAT, D_CAT), _np.float32)
for _h in range(HEADS):
    _EXPAND[_h, NH * _h:NH * (_h + 1)] = 1.0
    _EXPAND[DENW + _h, D_OUT + NH * _h:D_OUT + NH * (_h + 1)] = 1.0
    for _j in range(NH):
        _FOLD[NH * _h + _j, _j] = 1.0 / HEADS
        _FOLD[D_OUT + NH * _h + _j, NH + _j] = 1.0 / HEADS
_AVG[:NH, :NH] = 1.0 / NH
_AVG[NH:, NH:] = 1.0 / NH


def _sp(x):
    return jnp.maximum(x, 0.0) + jnp.log(1.0 + jnp.exp(-jnp.abs(x)))


def _embed_kernel(x_ref, w_ref, b_ref, o_ref):
    t = jnp.dot(x_ref[...], w_ref[...], preferred_element_type=jnp.float32)
    o_ref[...] = _sp(t + b_ref[...]).astype(jnp.bfloat16)


def _edge_embed_kernel(a_ref, w_ref, b_ref, o_ref):
    t = jnp.dot(a_ref[...], w_ref[...], preferred_element_type=jnp.float32)
    t = t + b_ref[...]
    o_ref[...] = jnp.maximum(t, 0.2 * t).astype(jnp.bfloat16)


def _qkv_kernel(xi_ref, xj_ref, ea_ref, wq_ref, wkv_ref, s_ref,
                v_ref, es_ref):
    ea = ea_ref[...]
    xi = xi_ref[...]
    xj = xj_ref[...]
    f32 = jnp.float32
    xiea0 = jnp.concatenate([xi[:, :NH], ea[:, :NH]], axis=1)
    xiea1 = jnp.concatenate([xi[:, NH:], ea[:, NH:]], axis=1)
    xjea0 = jnp.concatenate([xj[:, :NH], ea[:, :NH]], axis=1)
    xjea1 = jnp.concatenate([xj[:, NH:], ea[:, NH:]], axis=1)
    wq = wq_ref[...]
    wkv = wkv_ref[...]
    q0 = _sp(jnp.dot(xiea0, wq, preferred_element_type=f32))
    q1 = _sp(jnp.dot(xiea1, wq, preferred_element_type=f32))
    kv0 = _sp(jnp.dot(xjea0, wkv, preferred_element_type=f32))
    kv1 = _sp(jnp.dot(xjea1, wkv, preferred_element_type=f32))
    v_ref[...] = jnp.concatenate(
        [kv0[:, D_OUT:], kv1[:, D_OUT:]], axis=1).astype(jnp.bfloat16)
    sh = s_ref[...]
    s0 = jnp.dot(q0 * kv0[:, :D_OUT], sh, preferred_element_type=f32)
    s1 = jnp.dot(q1 * kv1[:, :D_OUT], sh, preferred_element_type=f32)
    es_ref[...] = jnp.exp(jnp.concatenate([s0, s1], axis=1))


def _edge_out_kernel(v_ref, es_ref, den_ref, xi_ref, b_ref,
                     ex_ref, fo_ref, av_ref, y_ref):
    f32 = jnp.float32
    al = es_ref[...] / (den_ref[...] + 1e-16)
    alx = jnp.dot(al, ex_ref[...], preferred_element_type=f32)
    va = v_ref[...].astype(f32) * alx
    out = jnp.dot(va, fo_ref[...], preferred_element_type=f32) + b_ref[...]
    av = av_ref[...]
    mu = jnp.dot(out, av, preferred_element_type=f32)
    d = out - mu
    var = jnp.dot(d * d, av, preferred_element_type=f32)
    out = d / jnp.sqrt(var + 1e-5)
    y_ref[...] = _sp(out + xi_ref[...].astype(f32))


def _bn_kernel(agg_ref, g_ref, b_ref, o_ref, ob_ref):
    out = agg_ref[0:N, :] + agg_ref[N_TAB:N_TAB + N, :]
    mu = jnp.mean(out, axis=0, keepdims=True)
    var = jnp.mean((out - mu) ** 2, axis=0, keepdims=True)
    hn = g_ref[...] * (out - mu) / jnp.sqrt(var + 1e-5) + b_ref[...]
    hp = jnp.concatenate(
        [hn, jnp.zeros((N_TAB - N, NH), jnp.float32)], axis=0)
    o_ref[...] = hp
    ob_ref[...] = hp.astype(jnp.bfloat16)


def _node_embed(x_pad, en_w, en_b):
    return pl.pallas_call(
        _embed_kernel,
        out_shape=jax.ShapeDtypeStruct((N_TAB, NH), jnp.bfloat16),
    )(x_pad, en_w, en_b.reshape(1, NH))


def _edge_embed(attr2, w2e, b2e):
    grid = 25
    rb = (E // 2) // grid
    return pl.pallas_call(
        _edge_embed_kernel,
        grid=(grid,),
        in_specs=[
            pl.BlockSpec((rb, 2 * F_EDGE), lambda i: (i, 0)),
            pl.BlockSpec((2 * F_EDGE, D_CAT), lambda i: (0, 0)),
            pl.BlockSpec((1, D_CAT), lambda i: (0, 0)),
        ],
        out_specs=pl.BlockSpec((rb, D_CAT), lambda i: (i, 0)),
        out_shape=jax.ShapeDtypeStruct((E_PAD // 2, D_CAT), jnp.bfloat16),
    )(attr2, w2e, b2e)


def _qkv_scores(xi_pack, xj_pack, ea_pack, w2q, w2kv, smat):
    grid = (E_PAD // 2) // EB2
    return pl.pallas_call(
        _qkv_kernel,
        grid=(grid,),
        in_specs=[
            pl.BlockSpec((EB2, D_CAT), lambda i: (i, 0)),
            pl.BlockSpec((EB2, D_CAT), lambda i: (i, 0)),
            pl.BlockSpec((EB2, D_CAT), lambda i: (i, 0)),
            pl.BlockSpec((D_CAT, D_OUT), lambda i: (0, 0)),
            pl.BlockSpec((D_CAT, 2 * D_OUT), lambda i: (0, 0)),
            pl.BlockSpec((D_OUT, DENW), lambda i: (0, 0)),
        ],
        out_specs=[
            pl.BlockSpec((EB2, 2 * D_OUT), lambda i: (i, 0)),
            pl.BlockSpec((EB2, 2 * DENW), lambda i: (i, 0)),
        ],
        out_shape=[
            jax.ShapeDtypeStruct((E_PAD // 2, 2 * D_OUT), jnp.bfloat16),
            jax.ShapeDtypeStruct((E_PAD // 2, 2 * DENW), jnp.float32),
        ],
    )(xi_pack, xj_pack, ea_pack, w2q, w2kv, smat)


def _edge_out(v, es, den2, xi_pack, b2, expand, fold, avg):
    grid = (E_PAD // 2) // EB2
    return pl.pallas_call(
        _edge_out_kernel,
        grid=(grid,),
        in_specs=[
            pl.BlockSpec((EB2, 2 * D_OUT), lambda i: (i, 0)),
            pl.BlockSpec((EB2, 2 * DENW), lambda i: (i, 0)),
            pl.BlockSpec((EB2, 2 * DENW), lambda i: (i, 0)),
            pl.BlockSpec((EB2, D_CAT), lambda i: (i, 0)),
            pl.BlockSpec((1, D_CAT), lambda i: (0, 0)),
            pl.BlockSpec((2 * DENW, 2 * D_OUT), lambda i: (0, 0)),
            pl.BlockSpec((2 * D_OUT, D_CAT), lambda i: (0, 0)),
            pl.BlockSpec((D_CAT, D_CAT), lambda i: (0, 0)),
        ],
        out_specs=pl.BlockSpec((EB2, D_CAT), lambda i: (i, 0)),
        out_shape=jax.ShapeDtypeStruct((E_PAD // 2, D_CAT), jnp.float32),
    )(v, es, den2, xi_pack, b2, expand, fold, avg)


def _batchnorm(agg2, g, b):
    return pl.pallas_call(
        _bn_kernel,
        out_shape=[
            jax.ShapeDtypeStruct((N_TAB, NH), jnp.float32),
            jax.ShapeDtypeStruct((N_TAB, NH), jnp.bfloat16),
        ],
    )(agg2, g.reshape(1, NH), b.reshape(1, NH))


def _sc_gather(h, seg2d, idx2d):
    @functools.partial(
        pl.kernel,
        out_type=[
            jax.ShapeDtypeStruct((E_PAD, NH), jnp.bfloat16),
            jax.ShapeDtypeStruct((E_PAD, NH), jnp.bfloat16),
        ],
        mesh=_MESH,
        compiler_params=_SC_PARAMS,
        scratch_types=[
            pltpu.VMEM_SHARED((N_TAB, NH), jnp.bfloat16),
            pltpu.VMEM((_ROWS_W, _GB), jnp.int32),
            pltpu.VMEM((_ROWS_W, _GB), jnp.int32),
            pltpu.VMEM((_K, _GB, NH), jnp.bfloat16),
            pltpu.VMEM((_K, _GB, NH), jnp.bfloat16),
            pltpu.SemaphoreType.DMA,
            pltpu.SemaphoreType.DMA,
            pltpu.SemaphoreType.DMA,
            pltpu.SemaphoreType.DMA,
        ],
    )
    def k(h_hbm, si_hbm, sj_hbm, oi_hbm, oj_hbm,
          table, ii, ij, rbi, rbj, sgi, sgj, ssi, ssj):
        s = lax.axis_index("s")
        c = lax.axis_index("c")
        wid = s * _NC + c
        r0 = wid * _ROWS_W
        stripe = N_TAB // _NS
        pltpu.sync_copy(h_hbm.at[pl.ds(s * stripe, stripe)],
                        table.at[pl.ds(s * stripe, stripe)])
        pltpu.sync_copy(si_hbm.at[pl.ds(r0, _ROWS_W)], ii)
        pltpu.sync_copy(sj_hbm.at[pl.ds(r0, _ROWS_W)], ij)
        plsc.subcore_barrier()

        def group(g, carry):
            t0 = g * _K

            @pl.when(g > 0)
            def _():
                for b in range(_K):
                    pltpu.make_async_copy(
                        rbi.at[b], oi_hbm.at[pl.ds(0, _GB)], ssi).wait()
                    pltpu.make_async_copy(
                        rbj.at[b], oj_hbm.at[pl.ds(0, _GB)], ssj).wait()

            ds = []
            for b in range(_K):
                t = t0 + b
                d1 = pltpu.async_copy(table.at[ii.at[t]], rbi.at[b], sgi)
                d2 = pltpu.async_copy(table.at[ij.at[t]], rbj.at[b], sgj)
                ds.append((d1, d2))
            for b in range(_K):
                t = t0 + b
                d1, d2 = ds[b]
                d1.wait()
                d2.wait()
                off = (r0 + t) * _GB
                pltpu.async_copy(rbi.at[b], oi_hbm.at[pl.ds(off, _GB)], ssi)
                pltpu.async_copy(rbj.at[b], oj_hbm.at[pl.ds(off, _GB)], ssj)
            return carry

        lax.fori_loop(0, _ROWS_W // _K, group, 0)
        for b in range(_K):
            pltpu.make_async_copy(
                rbi.at[b], oi_hbm.at[pl.ds(0, _GB)], ssi).wait()
            pltpu.make_async_copy(
                rbj.at[b], oj_hbm.at[pl.ds(0, _GB)], ssj).wait()

    return k(h, seg2d, idx2d)


def _sc_denom(es, seg2d, zeros_nw):
    rows_s = _ROWS // _NS

    @functools.partial(
        pl.kernel,
        out_type=jax.ShapeDtypeStruct((E_PAD, DENW), jnp.float32),
        mesh=_MESH,
        compiler_params=_SC_PARAMS,
        scratch_types=[
            pltpu.VMEM_SHARED((N_TAB, DENW), jnp.float32),
            pltpu.VMEM((rows_s, _GB), jnp.int32),
            pltpu.VMEM((_K, _GB, DENW), jnp.float32),
            pltpu.VMEM((_K, _GB, DENW), jnp.float32),
            pltpu.SemaphoreType.DMA,
            pltpu.SemaphoreType.DMA,
            pltpu.SemaphoreType.DMA,
            pltpu.SemaphoreType.DMA,
        ],
    )
    def k(es_hbm, si_hbm, z_hbm, out_hbm,
          table, ib, vb, gb, sv, ssc, sg, so):
        s = lax.axis_index("s")
        c = lax.axis_index("c")
        wid = s * _NC + c
        stripe = N_TAB // _NS
        pltpu.sync_copy(z_hbm.at[pl.ds(s * stripe, stripe)],
                        table.at[pl.ds(s * stripe, stripe)])
        pltpu.sync_copy(si_hbm.at[pl.ds(s * rows_s, rows_s)], ib)
        plsc.subcore_barrier()

        def sgroup(g, carry):
            t0 = g * _K

            @pl.when(g > 0)
            def _():
                for b in range(_K):
                    pltpu.make_async_copy(
                        vb.at[b], table.at[pl.ds(0, _GB)], ssc).wait()

            ds = []
            for b in range(_K):
                t = t0 + b
                off = (s * rows_s + t) * _GB
                ds.append(pltpu.async_copy(
                    es_hbm.at[pl.ds(off, _GB)], vb.at[b], sv))
            for b in range(_K):
                t = t0 + b
                ds[b].wait()
                pltpu.async_copy(vb.at[b], table.at[ib.at[t]], ssc, add=True)
            return carry

        lax.fori_loop(0, rows_s // _K, sgroup, 0)
        for b in range(_K):
            pltpu.make_async_copy(
                vb.at[b], table.at[pl.ds(0, _GB)], ssc).wait()
        plsc.subcore_barrier()

        def ggroup(g, carry):
            t0 = g * _K

            @pl.when(g > 0)
            def _():
                for b in range(_K):
                    pltpu.make_async_copy(
                        gb.at[b], out_hbm.at[pl.ds(0, _GB)], so).wait()

            ds = []
            for b in range(_K):
                t = t0 + b
                ds.append(pltpu.async_copy(
                    table.at[ib.at[c * _ROWS_W + t]], gb.at[b], sg))
            for b in range(_K):
                t = t0 + b
                ds[b].wait()
                off = (wid * _ROWS_W + t) * _GB
                pltpu.async_copy(gb.at[b], out_hbm.at[pl.ds(off, _GB)], so)
            return carry

        lax.fori_loop(0, _ROWS_W // _K, ggroup, 0)
        for b in range(_K):
            pltpu.make_async_copy(
                gb.at[b], out_hbm.at[pl.ds(0, _GB)], so).wait()

    return k(es, seg2d, zeros_nw)


def _sc_segsum(y, seg2d, zeros_n):
    rows_cs = _ROWS // _NW

    @functools.partial(
        pl.kernel,
        out_type=jax.ShapeDtypeStruct((2 * N_TAB, NH), jnp.float32),
        mesh=_MESH,
        compiler_params=_SC_PARAMS,
        scratch_types=[
            pltpu.VMEM_SHARED((N_TAB, NH), jnp.float32),
            pltpu.VMEM((rows_cs, _GB), jnp.int32),
            pltpu.VMEM((_K, _GB, NH), jnp.float32),
            pltpu.SemaphoreType.DMA,
            pltpu.SemaphoreType.DMA,
        ],
    )
    def k(y_hbm, si_hbm, z_hbm, out_hbm, table, ib, vb, sv, ssc):
        s = lax.axis_index("s")
        c = lax.axis_index("c")
        stripe = N_TAB // _NS
        r0 = c * (_ROWS // _NC) + s * rows_cs
        pltpu.sync_copy(z_hbm.at[pl.ds(s * stripe, stripe)],
                        table.at[pl.ds(s * stripe, stripe)])
        pltpu.sync_copy(si_hbm.at[pl.ds(r0, rows_cs)], ib)
        plsc.subcore_barrier()

        def sgroup(g, carry):
            t0 = g * _K

            @pl.when(g > 0)
            def _():
                for b in range(_K):
                    pltpu.make_async_copy(
                        vb.at[b], table.at[pl.ds(0, _GB)], ssc).wait()

            ds = []
            for b in range(_K):
                t = t0 + b
                off = (r0 + t) * _GB
                ds.append(pltpu.async_copy(
                    y_hbm.at[pl.ds(off, _GB)], vb.at[b], sv))
            for b in range(_K):
                t = t0 + b
                ds[b].wait()
                pltpu.async_copy(vb.at[b], table.at[ib.at[t]], ssc, add=True)
            return carry

        lax.fori_loop(0, rows_cs // _K, sgroup, 0)
        for b in range(_K):
            pltpu.make_async_copy(
                vb.at[b], table.at[pl.ds(0, _GB)], ssc).wait()
        plsc.subcore_barrier()

        pltpu.sync_copy(table.at[pl.ds(s * stripe, stripe)],
                        out_hbm.at[pl.ds(c * N_TAB + s * stripe, stripe)])

    return k(y, seg2d, zeros_n)


def kernel(x, edge_index, edge_attr, en_w, en_b, ee_w, ee_b, Wq, Wk, Wv,
           att_b, bn_g, bn_b):
    pad = jnp.full((E_PAD - E,), PAD_ROW, jnp.int32)
    seg2d = jnp.concatenate([edge_index[0], pad]).reshape(_ROWS, _GB)
    idx2d = jnp.concatenate([edge_index[1], pad]).reshape(_ROWS, _GB)
    x_pad = jnp.pad(x, ((0, N_TAB - N), (0, 0)))
    zeros_nw = jnp.zeros((N_TAB, DENW), jnp.float32)
    zeros_n = jnp.zeros((N_TAB, NH), jnp.float32)
    smat = jnp.asarray(_S_HALF)
    expand = jnp.asarray(_EXPAND)
    fold = jnp.asarray(_FOLD)
    avg = jnp.asarray(_AVG)

    attr2 = edge_attr.reshape(E // 2, 2 * F_EDGE)
    zf = jnp.zeros((F_EDGE, NH), jnp.float32)
    w2e = jnp.concatenate(
        [jnp.concatenate([ee_w, zf], axis=1),
         jnp.concatenate([zf, ee_w], axis=1)], axis=0)
    b2e = jnp.concatenate([ee_b, ee_b]).reshape(1, D_CAT)

    hb = _node_embed(x_pad, en_w, en_b)
    ea_pack = _edge_embed(attr2, w2e, b2e)
    for l in range(NL):
        wq = Wq[l].astype(jnp.bfloat16)
        wkv = jnp.concatenate([Wk[l], Wv[l]], axis=1).astype(jnp.bfloat16)
        b2 = jnp.concatenate([att_b[l], att_b[l]]).reshape(1, D_CAT)
        x_i, x_j = _sc_gather(hb, seg2d, idx2d)
        xi_pack = x_i.reshape(E_PAD // 2, D_CAT)
        xj_pack = x_j.reshape(E_PAD // 2, D_CAT)
        v, es = _qkv_scores(xi_pack, xj_pack, ea_pack, wq, wkv, smat)
        den_e = _sc_denom(es.reshape(E_PAD, DENW), seg2d, zeros_nw)
        y = _edge_out(v, es, den_e.reshape(E_PAD // 2, 2 * DENW),
                      xi_pack, b2, expand, fold, avg)
        agg2 = _sc_segsum(y.reshape(E_PAD, NH), seg2d, zeros_n)
        h, hb = _batchnorm(agg2, bn_g[l], bn_b[l])
    return h[:N]


# speedup vs baseline: 1.2821x; 1.2821x over previous
import functools
import math

import jax
import jax.numpy as jnp
from jax import lax
from jax.experimental import pallas as pl
from jax.experimental.pallas import tpu as pltpu
from jax.experimental.pallas import tpu_sc as plsc

N = 10000
E = 160000
HEADS = 4
F_IN = 92
F_EDGE = 41
NH = 64
NL = 3
D_CAT = NH + NH
D_OUT = HEADS * NH

_GB = 128
E_PAD = 1280 * _GB
N_TAB = N + 16
PAD_ROW = N
EB = 2048
DENW = 16

_NC = 2
_NS = 16
_NW = _NC * _NS
_ROWS = E_PAD // _GB
_ROWS_W = _ROWS // _NW
_K = 4

_MESH = plsc.VectorSubcoreMesh(core_axis_name="c", subcore_axis_name="s")
_SC_PARAMS = pltpu.CompilerParams(use_tc_tiling_on_sc=False)


EB2 = 1024

import numpy as _np

_S_HALF = _np.zeros((D_OUT, DENW), _np.float32)
for _h in range(HEADS):
    _S_HALF[NH * _h:NH * (_h + 1), _h] = 1.0 / math.sqrt(NH)

_EXPAND = _np.zeros((2 * DENW, 2 * D_OUT), _np.float32)
_FOLD = _np.zeros((2 * D_OUT, D_CAT), _np.float32)
_AVG = _np.zeros((D_CAT, D_CAT), _np.float32)
for _h in range(HEADS):
    _EXPAND[_h, NH * _h:NH * (_h + 1)] = 1.0
    _EXPAND[DENW + _h, D_OUT + NH * _h:D_OUT + NH * (_h + 1)] = 1.0
    for _j in range(NH):
        _FOLD[NH * _h + _j, _j] = 1.0 / HEADS
        _FOLD[D_OUT + NH * _h + _j, NH + _j] = 1.0 / HEADS
_AVG[:NH, :NH] = 1.0 / NH
_AVG[NH:, NH:] = 1.0 / NH


def _sp(x):
    return jnp.maximum(x, 0.0) + jnp.log(1.0 + jnp.exp(-jnp.abs(x)))


def _embed_kernel(x_ref, w_ref, b_ref, o_ref):
    t = jnp.dot(x_ref[...], w_ref[...], preferred_element_type=jnp.float32)
    o_ref[...] = _sp(t + b_ref[...])


def _edge_embed_kernel(a_ref, w_ref, b_ref, o_ref):
    t = jnp.dot(a_ref[...], w_ref[...], preferred_element_type=jnp.float32)
    t = t + b_ref[...]
    o_ref[...] = jnp.maximum(t, 0.2 * t).astype(jnp.bfloat16)


def _qkv_kernel(xi_ref, xj_ref, ea_ref, wq_ref, wkv_ref, s_ref,
                v_ref, es_ref):
    f32 = jnp.float32
    ea = ea_ref[...].astype(f32)
    xi = xi_ref[...]
    xj = xj_ref[...]
    xiea0 = jnp.concatenate([xi[:, :NH], ea[:, :NH]], axis=1)
    xiea1 = jnp.concatenate([xi[:, NH:], ea[:, NH:]], axis=1)
    xjea0 = jnp.concatenate([xj[:, :NH], ea[:, :NH]], axis=1)
    xjea1 = jnp.concatenate([xj[:, NH:], ea[:, NH:]], axis=1)
    wq = wq_ref[...]
    wkv = wkv_ref[...]
    q0 = _sp(jnp.dot(xiea0, wq, preferred_element_type=f32))
    q1 = _sp(jnp.dot(xiea1, wq, preferred_element_type=f32))
    kv0 = _sp(jnp.dot(xjea0, wkv, preferred_element_type=f32))
    kv1 = _sp(jnp.dot(xjea1, wkv, preferred_element_type=f32))
    v_ref[...] = jnp.concatenate(
        [kv0[:, D_OUT:], kv1[:, D_OUT:]], axis=1).astype(jnp.bfloat16)
    sh = s_ref[...]
    s0 = jnp.dot(q0 * kv0[:, :D_OUT], sh, preferred_element_type=f32)
    s1 = jnp.dot(q1 * kv1[:, :D_OUT], sh, preferred_element_type=f32)
    es_ref[...] = jnp.exp(jnp.concatenate([s0, s1], axis=1))


def _edge_out_kernel(v_ref, es_ref, den_ref, xi_ref, b_ref,
                     ex_ref, fo_ref, av_ref, y_ref):
    f32 = jnp.float32
    al = es_ref[...] / (den_ref[...] + 1e-16)
    alx = jnp.dot(al, ex_ref[...], preferred_element_type=f32)
    va = v_ref[...].astype(f32) * alx
    out = jnp.dot(va, fo_ref[...], preferred_element_type=f32) + b_ref[...]
    av = av_ref[...]
    mu = jnp.dot(out, av, preferred_element_type=f32)
    d = out - mu
    var = jnp.dot(d * d, av, preferred_element_type=f32)
    out = d / jnp.sqrt(var + 1e-5)
    y_ref[...] = _sp(out + xi_ref[...].astype(f32))


def _bn_kernel(agg_ref, g_ref, b_ref, o_ref):
    out = agg_ref[0:N, :] + agg_ref[N_TAB:N_TAB + N, :]
    mu = jnp.mean(out, axis=0, keepdims=True)
    var = jnp.mean((out - mu) ** 2, axis=0, keepdims=True)
    hn = g_ref[...] * (out - mu) / jnp.sqrt(var + 1e-5) + b_ref[...]
    o_ref[...] = jnp.concatenate(
        [hn, jnp.zeros((N_TAB - N, NH), jnp.float32)], axis=0)


def _node_embed(x_pad, en_w, en_b):
    return pl.pallas_call(
        _embed_kernel,
        out_shape=jax.ShapeDtypeStruct((N_TAB, NH), jnp.float32),
    )(x_pad, en_w, en_b.reshape(1, NH))


def _edge_embed(attr2, w2e, b2e):
    grid = 25
    rb = (E // 2) // grid
    return pl.pallas_call(
        _edge_embed_kernel,
        grid=(grid,),
        in_specs=[
            pl.BlockSpec((rb, 2 * F_EDGE), lambda i: (i, 0)),
            pl.BlockSpec((2 * F_EDGE, D_CAT), lambda i: (0, 0)),
            pl.BlockSpec((1, D_CAT), lambda i: (0, 0)),
        ],
        out_specs=pl.BlockSpec((rb, D_CAT), lambda i: (i, 0)),
        out_shape=jax.ShapeDtypeStruct((E_PAD // 2, D_CAT), jnp.bfloat16),
    )(attr2, w2e, b2e)


def _qkv_scores(xi_pack, xj_pack, ea_pack, w2q, w2kv, smat):
    grid = (E_PAD // 2) // EB2
    return pl.pallas_call(
        _qkv_kernel,
        grid=(grid,),
        in_specs=[
            pl.BlockSpec((EB2, D_CAT), lambda i: (i, 0)),
            pl.BlockSpec((EB2, D_CAT), lambda i: (i, 0)),
            pl.BlockSpec((EB2, D_CAT), lambda i: (i, 0)),
            pl.BlockSpec((D_CAT, D_OUT), lambda i: (0, 0)),
            pl.BlockSpec((D_CAT, 2 * D_OUT), lambda i: (0, 0)),
            pl.BlockSpec((D_OUT, DENW), lambda i: (0, 0)),
        ],
        out_specs=[
            pl.BlockSpec((EB2, 2 * D_OUT), lambda i: (i, 0)),
            pl.BlockSpec((EB2, 2 * DENW), lambda i: (i, 0)),
        ],
        out_shape=[
            jax.ShapeDtypeStruct((E_PAD // 2, 2 * D_OUT), jnp.bfloat16),
            jax.ShapeDtypeStruct((E_PAD // 2, 2 * DENW), jnp.float32),
        ],
    )(xi_pack, xj_pack, ea_pack, w2q, w2kv, smat)


def _edge_out(v, es, den2, xi_pack, b2, expand, fold, avg):
    grid = (E_PAD // 2) // EB2
    return pl.pallas_call(
        _edge_out_kernel,
        grid=(grid,),
        in_specs=[
            pl.BlockSpec((EB2, 2 * D_OUT), lambda i: (i, 0)),
            pl.BlockSpec((EB2, 2 * DENW), lambda i: (i, 0)),
            pl.BlockSpec((EB2, 2 * DENW), lambda i: (i, 0)),
            pl.BlockSpec((EB2, D_CAT), lambda i: (i, 0)),
            pl.BlockSpec((1, D_CAT), lambda i: (0, 0)),
            pl.BlockSpec((2 * DENW, 2 * D_OUT), lambda i: (0, 0)),
            pl.BlockSpec((2 * D_OUT, D_CAT), lambda i: (0, 0)),
            pl.BlockSpec((D_CAT, D_CAT), lambda i: (0, 0)),
        ],
        out_specs=pl.BlockSpec((EB2, D_CAT), lambda i: (i, 0)),
        out_shape=jax.ShapeDtypeStruct((E_PAD // 2, D_CAT), jnp.float32),
    )(v, es, den2, xi_pack, b2, expand, fold, avg)


def _batchnorm(agg2, g, b):
    return pl.pallas_call(
        _bn_kernel,
        out_shape=jax.ShapeDtypeStruct((N_TAB, NH), jnp.float32),
    )(agg2, g.reshape(1, NH), b.reshape(1, NH))


def _sc_gather(h, seg2d, idx2d):
    @functools.partial(
        pl.kernel,
        out_type=[
            jax.ShapeDtypeStruct((E_PAD, NH), jnp.float32),
            jax.ShapeDtypeStruct((E_PAD, NH), jnp.float32),
        ],
        mesh=_MESH,
        compiler_params=_SC_PARAMS,
        scratch_types=[
            pltpu.VMEM_SHARED((N_TAB, NH), jnp.float32),
            pltpu.VMEM((_ROWS_W, _GB), jnp.int32),
            pltpu.VMEM((_ROWS_W, _GB), jnp.int32),
            pltpu.VMEM((_K, _GB, NH), jnp.float32),
            pltpu.VMEM((_K, _GB, NH), jnp.float32),
            pltpu.SemaphoreType.DMA,
            pltpu.SemaphoreType.DMA,
            pltpu.SemaphoreType.DMA,
            pltpu.SemaphoreType.DMA,
        ],
    )
    def k(h_hbm, si_hbm, sj_hbm, oi_hbm, oj_hbm,
          table, ii, ij, rbi, rbj, sgi, sgj, ssi, ssj):
        s = lax.axis_index("s")
        c = lax.axis_index("c")
        wid = s * _NC + c
        r0 = wid * _ROWS_W
        stripe = N_TAB // _NS
        pltpu.sync_copy(h_hbm.at[pl.ds(s * stripe, stripe)],
                        table.at[pl.ds(s * stripe, stripe)])
        pltpu.sync_copy(si_hbm.at[pl.ds(r0, _ROWS_W)], ii)
        pltpu.sync_copy(sj_hbm.at[pl.ds(r0, _ROWS_W)], ij)
        plsc.subcore_barrier()

        def group(g, carry):
            t0 = g * _K

            @pl.when(g > 0)
            def _():
                for b in range(_K):
                    pltpu.make_async_copy(
                        rbi.at[b], oi_hbm.at[pl.ds(0, _GB)], ssi).wait()
                    pltpu.make_async_copy(
                        rbj.at[b], oj_hbm.at[pl.ds(0, _GB)], ssj).wait()

            ds = []
            for b in range(_K):
                t = t0 + b
                d1 = pltpu.async_copy(table.at[ii.at[t]], rbi.at[b], sgi)
                d2 = pltpu.async_copy(table.at[ij.at[t]], rbj.at[b], sgj)
                ds.append((d1, d2))
            for b in range(_K):
                t = t0 + b
                d1, d2 = ds[b]
                d1.wait()
                d2.wait()
                off = (r0 + t) * _GB
                pltpu.async_copy(rbi.at[b], oi_hbm.at[pl.ds(off, _GB)], ssi)
                pltpu.async_copy(rbj.at[b], oj_hbm.at[pl.ds(off, _GB)], ssj)
            return carry

        lax.fori_loop(0, _ROWS_W // _K, group, 0)
        for b in range(_K):
            pltpu.make_async_copy(
                rbi.at[b], oi_hbm.at[pl.ds(0, _GB)], ssi).wait()
            pltpu.make_async_copy(
                rbj.at[b], oj_hbm.at[pl.ds(0, _GB)], ssj).wait()

    return k(h, seg2d, idx2d)


def _sc_denom(es, seg2d, zeros_nw):
    rows_s = _ROWS // _NS

    @functools.partial(
        pl.kernel,
        out_type=jax.ShapeDtypeStruct((E_PAD, DENW), jnp.float32),
        mesh=_MESH,
        compiler_params=_SC_PARAMS,
        scratch_types=[
            pltpu.VMEM_SHARED((N_TAB, DENW), jnp.float32),
            pltpu.VMEM((rows_s, _GB), jnp.int32),
            pltpu.VMEM((_K, _GB, DENW), jnp.float32),
            pltpu.VMEM((_K, _GB, DENW), jnp.float32),
            pltpu.SemaphoreType.DMA,
            pltpu.SemaphoreType.DMA,
            pltpu.SemaphoreType.DMA,
            pltpu.SemaphoreType.DMA,
        ],
    )
    def k(es_hbm, si_hbm, z_hbm, out_hbm,
          table, ib, vb, gb, sv, ssc, sg, so):
        s = lax.axis_index("s")
        c = lax.axis_index("c")
        wid = s * _NC + c
        stripe = N_TAB // _NS
        pltpu.sync_copy(z_hbm.at[pl.ds(s * stripe, stripe)],
                        table.at[pl.ds(s * stripe, stripe)])
        pltpu.sync_copy(si_hbm.at[pl.ds(s * rows_s, rows_s)], ib)
        plsc.subcore_barrier()

        def sgroup(g, carry):
            t0 = g * _K

            @pl.when(g > 0)
            def _():
                for b in range(_K):
                    pltpu.make_async_copy(
                        vb.at[b], table.at[pl.ds(0, _GB)], ssc).wait()

            ds = []
            for b in range(_K):
                t = t0 + b
                off = (s * rows_s + t) * _GB
                ds.append(pltpu.async_copy(
                    es_hbm.at[pl.ds(off, _GB)], vb.at[b], sv))
            for b in range(_K):
                t = t0 + b
                ds[b].wait()
                pltpu.async_copy(vb.at[b], table.at[ib.at[t]], ssc, add=True)
            return carry

        lax.fori_loop(0, rows_s // _K, sgroup, 0)
        for b in range(_K):
            pltpu.make_async_copy(
                vb.at[b], table.at[pl.ds(0, _GB)], ssc).wait()
        plsc.subcore_barrier()

        def ggroup(g, carry):
            t0 = g * _K

            @pl.when(g > 0)
            def _():
                for b in range(_K):
                    pltpu.make_async_copy(
                        gb.at[b], out_hbm.at[pl.ds(0, _GB)], so).wait()

            ds = []
            for b in range(_K):
                t = t0 + b
                ds.append(pltpu.async_copy(
                    table.at[ib.at[c * _ROWS_W + t]], gb.at[b], sg))
            for b in range(_K):
                t = t0 + b
                ds[b].wait()
                off = (wid * _ROWS_W + t) * _GB
                pltpu.async_copy(gb.at[b], out_hbm.at[pl.ds(off, _GB)], so)
            return carry

        lax.fori_loop(0, _ROWS_W // _K, ggroup, 0)
        for b in range(_K):
            pltpu.make_async_copy(
                gb.at[b], out_hbm.at[pl.ds(0, _GB)], so).wait()

    return k(es, seg2d, zeros_nw)


def _sc_segsum(y, seg2d, zeros_n):
    rows_cs = _ROWS // _NW

    @functools.partial(
        pl.kernel,
        out_type=jax.ShapeDtypeStruct((2 * N_TAB, NH), jnp.float32),
        mesh=_MESH,
        compiler_params=_SC_PARAMS,
        scratch_types=[
            pltpu.VMEM_SHARED((N_TAB, NH), jnp.float32),
            pltpu.VMEM((rows_cs, _GB), jnp.int32),
            pltpu.VMEM((_K, _GB, NH), jnp.float32),
            pltpu.SemaphoreType.DMA,
            pltpu.SemaphoreType.DMA,
        ],
    )
    def k(y_hbm, si_hbm, z_hbm, out_hbm, table, ib, vb, sv, ssc):
        s = lax.axis_index("s")
        c = lax.axis_index("c")
        stripe = N_TAB // _NS
        r0 = c * (_ROWS // _NC) + s * rows_cs
        pltpu.sync_copy(z_hbm.at[pl.ds(s * stripe, stripe)],
                        table.at[pl.ds(s * stripe, stripe)])
        pltpu.sync_copy(si_hbm.at[pl.ds(r0, rows_cs)], ib)
        plsc.subcore_barrier()

        def sgroup(g, carry):
            t0 = g * _K

            @pl.when(g > 0)
            def _():
                for b in range(_K):
                    pltpu.make_async_copy(
                        vb.at[b], table.at[pl.ds(0, _GB)], ssc).wait()

            ds = []
            for b in range(_K):
                t = t0 + b
                off = (r0 + t) * _GB
                ds.append(pltpu.async_copy(
                    y_hbm.at[pl.ds(off, _GB)], vb.at[b], sv))
            for b in range(_K):
                t = t0 + b
                ds[b].wait()
                pltpu.async_copy(vb.at[b], table.at[ib.at[t]], ssc, add=True)
            return carry

        lax.fori_loop(0, rows_cs // _K, sgroup, 0)
        for b in range(_K):
            pltpu.make_async_copy(
                vb.at[b], table.at[pl.ds(0, _GB)], ssc).wait()
        plsc.subcore_barrier()

        pltpu.sync_copy(table.at[pl.ds(s * stripe, stripe)],
                        out_hbm.at[pl.ds(c * N_TAB + s * stripe, stripe)])

    return k(y, seg2d, zeros_n)


def kernel(x, edge_index, edge_attr, en_w, en_b, ee_w, ee_b, Wq, Wk, Wv,
           att_b, bn_g, bn_b):
    pad = jnp.full((E_PAD - E,), PAD_ROW, jnp.int32)
    seg2d = jnp.concatenate([edge_index[0], pad]).reshape(_ROWS, _GB)
    idx2d = jnp.concatenate([edge_index[1], pad]).reshape(_ROWS, _GB)
    x_pad = jnp.pad(x, ((0, N_TAB - N), (0, 0)))
    zeros_nw = jnp.zeros((N_TAB, DENW), jnp.float32)
    zeros_n = jnp.zeros((N_TAB, NH), jnp.float32)
    smat = jnp.asarray(_S_HALF)
    expand = jnp.asarray(_EXPAND)
    fold = jnp.asarray(_FOLD)
    avg = jnp.asarray(_AVG)

    attr2 = edge_attr.reshape(E // 2, 2 * F_EDGE)
    zf = jnp.zeros((F_EDGE, NH), jnp.float32)
    w2e = jnp.concatenate(
        [jnp.concatenate([ee_w, zf], axis=1),
         jnp.concatenate([zf, ee_w], axis=1)], axis=0)
    b2e = jnp.concatenate([ee_b, ee_b]).reshape(1, D_CAT)

    h = _node_embed(x_pad, en_w, en_b)
    ea_pack = _edge_embed(attr2, w2e, b2e)
    for l in range(NL):
        wq = Wq[l]
        wkv = jnp.concatenate([Wk[l], Wv[l]], axis=1)
        b2 = jnp.concatenate([att_b[l], att_b[l]]).reshape(1, D_CAT)
        x_i, x_j = _sc_gather(h, seg2d, idx2d)
        xi_pack = x_i.reshape(E_PAD // 2, D_CAT)
        xj_pack = x_j.reshape(E_PAD // 2, D_CAT)
        v, es = _qkv_scores(xi_pack, xj_pack, ea_pack, wq, wkv, smat)
        den_e = _sc_denom(es.reshape(E_PAD, DENW), seg2d, zeros_nw)
        y = _edge_out(v, es, den_e.reshape(E_PAD // 2, 2 * DENW),
                      xi_pack, b2, expand, fold, avg)
        agg2 = _sc_segsum(y.reshape(E_PAD, NH), seg2d, zeros_n)
        h = _batchnorm(agg2, bn_g[l], bn_b[l])
    return h[:N]
